# baseline (speedup 1.0000x reference)
import functools

import jax
import jax.numpy as jnp
from jax import lax
from jax.experimental import pallas as pl
from jax.experimental.pallas import tpu as pltpu
from jax.experimental.pallas import tpu_sc as plsc

D = 64
DE = 16
DEH = 64
DD = D * D
STEPS = 3
N_LIG, E_LIG = 800, 6400
N_REC, E_REC = 3200, 25600
NEG = 0.01

N_REAL = N_LIG + N_REC
N_ALL = 4096
E_REAL = E_LIG + E_REC
E_ALL = 32768
PAD_ROW = N_ALL - 1
D2 = 128

NC, NS = 2, 16
NW = NC * NS
CHUNK = 128
RPS = N_ALL // NS

EA = E_ALL // 2
EB = E_ALL - EA
NCHA = EA // NW // CHUNK
NCHB = EB // NW // CHUNK

ET = 512


def _lrelu(x):
    return jnp.where(x > 0, x, NEG * x)


def _ln(x, g, b, eps=1e-5):
    m = jnp.mean(x, axis=-1, keepdims=True)
    v = jnp.mean((x - m) ** 2, axis=-1, keepdims=True)
    return (x - m) * lax.rsqrt(v + eps) * g + b


def _pre_body(nl_ref, nr_ref, el_ref, er_ref, w0_ref, b0_ref, we1_ref,
              be1c_ref, h_ref, zt_ref):
    w0 = w0_ref[...]
    b0 = b0_ref[...]
    zcol = jnp.zeros((N_LIG, D2 - D), jnp.float32)
    hl = jnp.maximum(jnp.dot(nl_ref[...], w0,
                             preferred_element_type=jnp.float32) + b0, 0.0)
    h_ref[0:N_LIG, :] = jnp.concatenate([hl, zcol], axis=1)
    hr = jnp.maximum(jnp.dot(nr_ref[...], w0,
                             preferred_element_type=jnp.float32) + b0, 0.0)
    h_ref[N_LIG:N_REAL, :] = jnp.concatenate(
        [hr, jnp.zeros((N_REC, D2 - D), jnp.float32)], axis=1)
    h_ref[N_REAL:N_ALL, :] = jnp.zeros((N_ALL - N_REAL, D2), jnp.float32)
    we1 = we1_ref[...]
    be1c = be1c_ref[...]
    dn = (((0,), (1,)), ((), ()))
    zl = jnp.maximum(lax.dot_general(we1, el_ref[...], dn,
                                     preferred_element_type=jnp.float32)
                     + be1c, 0.0)
    zt_ref[:, 0:E_LIG] = zl.astype(jnp.bfloat16)
    zr = jnp.maximum(lax.dot_general(we1, er_ref[...], dn,
                                     preferred_element_type=jnp.float32)
                     + be1c, 0.0)
    zt_ref[:, E_LIG:E_REAL] = zr.astype(jnp.bfloat16)
    zt_ref[:, E_REAL:E_ALL] = jnp.zeros((DEH, E_ALL - E_REAL), jnp.bfloat16)


def _pre(nl, nr, el, er, w0, b0, we1, be1c):
    return pl.pallas_call(
        _pre_body,
        out_shape=(
            jax.ShapeDtypeStruct((N_ALL, D2), jnp.float32),
            jax.ShapeDtypeStruct((DEH, E_ALL), jnp.bfloat16),
        ),
    )(nl, nr, el, er, w0, b0, we1, be1c)


def _msg_body(zt_ref, g_ref, we2t_ref, be2t_ref, msg_ref):
    ewt = jnp.dot(we2t_ref[...], zt_ref[...],
                  preferred_element_type=jnp.float32)
    gt = jnp.transpose(g_ref[:, :D])
    acc = jnp.dot(be2t_ref[...], gt, preferred_element_type=jnp.float32)
    for i in range(D):
        acc = acc + gt[i:i + 1, :] * ewt[i * D:(i + 1) * D, :]
    out = jnp.transpose(acc)
    msg_ref[...] = jnp.concatenate(
        [out, jnp.zeros((ET, D2 - D), jnp.float32)], axis=1)


def _msg(zt, g, we2t, be2t, off):
    blk_off = off // ET
    ne = g.shape[0]
    return pl.pallas_call(
        _msg_body,
        grid=(ne // ET,),
        in_specs=[
            pl.BlockSpec((DEH, ET), lambda i: (0, i + blk_off)),
            pl.BlockSpec((ET, D2), lambda i: (i, 0)),
            pl.BlockSpec((DD, DEH), lambda i: (0, 0)),
            pl.BlockSpec((D, D), lambda i: (0, 0)),
        ],
        out_specs=pl.BlockSpec((ET, D2), lambda i: (i, 0)),
        out_shape=jax.ShapeDtypeStruct((ne, D2), jnp.float32),
    )(zt, g, we2t, be2t)


def _upd_body(pa_ref, pb_ref, h_ref, cb_ref, wm_ref, bm_ref, out_ref):
    agg = (pa_ref[0][:, :D] + pa_ref[1][:, :D]
           + pb_ref[0][:, :D] + pb_ref[1][:, :D])
    h = h_ref[:, :D]
    m = jnp.maximum(agg + h + cb_ref[...], 0.0)
    hn = (
        jnp.dot(m, wm_ref[0], preferred_element_type=jnp.float32)
        + jnp.dot(h, wm_ref[1], preferred_element_type=jnp.float32)
        + bm_ref[...]
    )
    out_ref[...] = jnp.concatenate(
        [hn, jnp.zeros((N_ALL, D2 - D), jnp.float32)], axis=1)


def _update(parts_a, parts_b, h, cb, wm2, bm):
    return pl.pallas_call(
        _upd_body,
        out_shape=jax.ShapeDtypeStruct((N_ALL, D2), jnp.float32),
        input_output_aliases={2: 0},
    )(parts_a, parts_b, h, cb, wm2, bm)


def _make_att(q0, q1, k0, k1, flip=False):
    def _att_body(h_ref, nq_ref, nk_ref, mask_ref,
                  lnq_g_ref, lnq_b_ref, lnk_g_ref, lnk_b_ref,
                  wq_ref, wk_ref, wv_ref,
                  w1_ref, b1_ref, g1_ref, bn1_ref,
                  w2_ref, b2_ref, g2_ref, bn2_ref,
                  out_ref):
        hq = _ln(h_ref[q0:q1, :D] + nq_ref[...], lnq_g_ref[...],
                 lnq_b_ref[...])
        hk = _ln(h_ref[k0:k1, :D] + nk_ref[...], lnk_g_ref[...],
                 lnk_b_ref[...])
        q = _lrelu(jnp.dot(hq, wq_ref[...],
                           preferred_element_type=jnp.float32))
        k = _lrelu(jnp.dot(hk, wk_ref[...],
                           preferred_element_type=jnp.float32))
        v = jnp.dot(hk, wv_ref[...], preferred_element_type=jnp.float32)
        mask = mask_ref[...]
        if flip:
            s = lax.dot_general(k, q, (((1,), (1,)), ((), ())),
                                preferred_element_type=jnp.float32)
            a = mask * s - 1000.0 * (1.0 - mask)
            a = a - jnp.max(a, axis=0, keepdims=True)
            e = jnp.exp(a)
            prob = e / jnp.sum(e, axis=0, keepdims=True)
            att = lax.dot_general(prob, v, (((0,), (0,)), ((), ())),
                                  preferred_element_type=jnp.float32)
        else:
            s = lax.dot_general(q, k, (((1,), (1,)), ((), ())),
                                preferred_element_type=jnp.float32)
            a = mask * s - 1000.0 * (1.0 - mask)
            a = a - jnp.max(a, axis=1, keepdims=True)
            e = jnp.exp(a)
            prob = e / jnp.sum(e, axis=1, keepdims=True)
            att = jnp.dot(prob, v, preferred_element_type=jnp.float32)
        x = hq + att
        y = _lrelu(_ln(jnp.dot(x, w1_ref[...],
                               preferred_element_type=jnp.float32)
                       + b1_ref[...], g1_ref[...], bn1_ref[...]))
        out_ref[...] = _ln(jnp.dot(y, w2_ref[...],
                                   preferred_element_type=jnp.float32)
                           + b2_ref[...], g2_ref[...], bn2_ref[...])

    def call(h, nq, nk, mask, lnq_g, lnq_b, lnk_g, lnk_b, wq, wk, wv, mp):
        r1 = lambda a: a.reshape(1, -1)
        return pl.pallas_call(
            _att_body,
            out_shape=jax.ShapeDtypeStruct((q1 - q0, D), jnp.float32),
        )(h, nq, nk, mask,
          r1(lnq_g), r1(lnq_b), r1(lnk_g), r1(lnk_b),
          wq, wk, wv,
          mp['W1'], r1(mp['b1']), r1(mp['g1']), r1(mp['bn1']),
          mp['W2'], r1(mp['b2']), r1(mp['g2']), r1(mp['bn2']))

    return call


_att_lig = _make_att(0, N_LIG, N_LIG, N_REAL)
_att_rec = _make_att(N_LIG, N_REAL, 0, N_LIG, flip=True)


def _sc_mesh():
    return plsc.VectorSubcoreMesh(core_axis_name="c", subcore_axis_name="s",
                                  num_cores=NC, num_subcores=NS)


def _make_gather(nchunk):
    epw = nchunk * CHUNK

    def body(h_hbm, idx_hbm, out_hbm, idx_v, rows_v, sem):
        wid = lax.axis_index("s") * NC + lax.axis_index("c")
        pltpu.sync_copy(idx_hbm.at[wid], idx_v)
        copies = []
        for ci in range(nchunk):
            copies.append(pltpu.async_copy(
                h_hbm.at[idx_v.at[ci]],
                rows_v.at[pl.ds(ci * CHUNK, CHUNK)], sem))
        for cp in copies:
            cp.wait()
        pltpu.sync_copy(rows_v, out_hbm.at[pl.ds(wid * epw, epw)])

    def call(h, idx3):
        return pl.kernel(
            body,
            out_type=jax.ShapeDtypeStruct((NW * epw, D2), jnp.float32),
            mesh=_sc_mesh(),
            scratch_types=[
                pltpu.VMEM((nchunk, CHUNK), jnp.int32),
                pltpu.VMEM((epw, D2), jnp.float32),
                pltpu.SemaphoreType.DMA,
            ],
        )(h, idx3)

    return call


def _make_scatter(nchunk):
    epw = nchunk * CHUNK
    zrows = min(epw, RPS)

    def body(msg_hbm, dst_hbm, out_hbm, idx_v, rows_v, accum, sem):
        cid = lax.axis_index("c")
        sid = lax.axis_index("s")
        wid = sid * NC + cid
        pltpu.sync_copy(dst_hbm.at[wid], idx_v)
        zeros16 = jnp.zeros((16,), jnp.float32)

        @pl.loop(0, zrows)
        def _(i):
            for j in range(D2 // 16):
                rows_v[i, pl.ds(j * 16, 16)] = zeros16

        for t in range(RPS // zrows):
            pltpu.sync_copy(rows_v.at[pl.ds(0, zrows)],
                            accum.at[pl.ds(sid * RPS + t * zrows, zrows)])
        pltpu.sync_copy(msg_hbm.at[pl.ds(wid * epw, epw)], rows_v)
        plsc.subcore_barrier()
        for ci in range(nchunk):
            pltpu.sync_copy(rows_v.at[pl.ds(ci * CHUNK, CHUNK)],
                            accum.at[idx_v.at[ci]], add=True)
        plsc.subcore_barrier()
        pltpu.sync_copy(accum.at[pl.ds(sid * RPS, RPS)],
                        out_hbm.at[cid].at[pl.ds(sid * RPS, RPS)])

    def call(msg, dst3):
        return pl.kernel(
            body,
            out_type=jax.ShapeDtypeStruct((NC, N_ALL, D2), jnp.float32),
            mesh=_sc_mesh(),
            scratch_types=[
                pltpu.VMEM((nchunk, CHUNK), jnp.int32),
                pltpu.VMEM((epw, D2), jnp.float32),
                pltpu.VMEM_SHARED((N_ALL, D2), jnp.float32),
                pltpu.SemaphoreType.DMA,
            ],
        )(msg, dst3)

    return call


_GATHERS = {NCHA: _make_gather(NCHA), NCHB: _make_gather(NCHB)}
_SCATTERS = {NCHA: _make_scatter(NCHA), NCHB: _make_scatter(NCHB)}


def _sc_gather(h, idx3):
    return _GATHERS[idx3.shape[1]](h, idx3)


def _sc_scatter(msg, dst3):
    return _SCATTERS[dst3.shape[1]](msg, dst3)


def kernel(lig_n_feat, lig_e_feat, rec_n_feat, rec_e_feat, mask, params,
           lig_edge_index, rec_edge_index):
    p = params

    pad_idx = jnp.full((E_ALL - E_REAL,), PAD_ROW, jnp.int32)
    src = jnp.concatenate([lig_edge_index[0], rec_edge_index[0] + N_LIG,
                           pad_idx])
    dst = jnp.concatenate([lig_edge_index[1], rec_edge_index[1] + N_LIG,
                           pad_idx])
    src3a = src[:EA].reshape(NW, NCHA, CHUNK)
    src3b = src[EA:].reshape(NW, NCHB, CHUNK)
    dst3a = dst[:EA].reshape(NW, NCHA, CHUNK)
    dst3b = dst[EA:].reshape(NW, NCHB, CHUNK)

    r1 = lambda a: a.reshape(1, -1)
    wm2 = p['Wm'].reshape(2, D, D)
    we2t = p['We2'].T.astype(jnp.bfloat16)
    be2t = p['be2'].reshape(D, D).T

    h, zt = _pre(lig_n_feat, rec_n_feat, lig_e_feat, rec_e_feat,
                 p['W0'], r1(p['b0']), p['We1'], p['be1'].reshape(-1, 1))
    for _ in range(STEPS):
        g_a = _sc_gather(h, src3a)
        g_b = _sc_gather(h, src3b)
        msg_a = _msg(zt, g_a, we2t, be2t, 0)
        msg_b = _msg(zt, g_b, we2t, be2t, EA)
        parts_a = _sc_scatter(msg_a, dst3a)
        parts_b = _sc_scatter(msg_b, dst3b)
        h = _update(parts_a, parts_b, h, r1(p['conv_bias']), wm2, r1(p['bm']))

    out_lig = _att_lig(h, lig_n_feat, rec_n_feat, mask,
                       p['ln_lig_g'], p['ln_lig_b'], p['ln_rec_g'],
                       p['ln_rec_b'], p['Q_lig'], p['K_rec'], p['V_rec'],
                       p['mlp_lig'])
    out_rec = _att_rec(h, rec_n_feat, lig_n_feat, mask,
                       p['ln_rec_g'], p['ln_rec_b'], p['ln_lig_g'],
                       p['ln_lig_b'], p['Q_rec'], p['K_lig'], p['V_lig'],
                       p['mlp_rec'])
    return out_lig, out_rec


# speedup vs baseline: 1.0072x; 1.0072x over previous
import functools

import jax
import jax.numpy as jnp
from jax import lax
from jax.experimental import pallas as pl
from jax.experimental.pallas import tpu as pltpu
from jax.experimental.pallas import tpu_sc as plsc

D = 64
DE = 16
DEH = 64
DD = D * D
STEPS = 3
N_LIG, E_LIG = 800, 6400
N_REC, E_REC = 3200, 25600
NEG = 0.01

N_REAL = N_LIG + N_REC
N_ALL = 4096
E_REAL = E_LIG + E_REC
E_ALL = 32768
PAD_ROW = N_ALL - 1
D2 = 128

NC, NS = 2, 16
NW = NC * NS
CHUNK = 128
RPS = N_ALL // NS

EA = E_ALL // 2
EB = E_ALL - EA
NCHA = EA // NW // CHUNK
NCHB = EB // NW // CHUNK

ET = 512


def _lrelu(x):
    return jnp.where(x > 0, x, NEG * x)


def _ln(x, g, b, eps=1e-5):
    m = jnp.mean(x, axis=-1, keepdims=True)
    v = jnp.mean((x - m) ** 2, axis=-1, keepdims=True)
    return (x - m) * lax.rsqrt(v + eps) * g + b


def _pre_body(nl_ref, nr_ref, el_ref, er_ref, w0_ref, b0_ref, we1_ref,
              be1c_ref, h_ref, zt_ref):
    w0 = w0_ref[...]
    b0 = b0_ref[...]
    zcol = jnp.zeros((N_LIG, D2 - D), jnp.float32)
    hl = jnp.maximum(jnp.dot(nl_ref[...], w0,
                             preferred_element_type=jnp.float32) + b0, 0.0)
    h_ref[0:N_LIG, :] = jnp.concatenate([hl, zcol], axis=1)
    hr = jnp.maximum(jnp.dot(nr_ref[...], w0,
                             preferred_element_type=jnp.float32) + b0, 0.0)
    h_ref[N_LIG:N_REAL, :] = jnp.concatenate(
        [hr, jnp.zeros((N_REC, D2 - D), jnp.float32)], axis=1)
    h_ref[N_REAL:N_ALL, :] = jnp.zeros((N_ALL - N_REAL, D2), jnp.float32)
    we1 = we1_ref[...]
    be1c = be1c_ref[...]
    dn = (((0,), (1,)), ((), ()))
    zl = jnp.maximum(lax.dot_general(we1, el_ref[...], dn,
                                     preferred_element_type=jnp.float32)
                     + be1c, 0.0)
    zt_ref[:, 0:E_LIG] = zl.astype(jnp.bfloat16)
    zr = jnp.maximum(lax.dot_general(we1, er_ref[...], dn,
                                     preferred_element_type=jnp.float32)
                     + be1c, 0.0)
    zt_ref[:, E_LIG:E_REAL] = zr.astype(jnp.bfloat16)
    zt_ref[:, E_REAL:E_ALL] = jnp.zeros((DEH, E_ALL - E_REAL), jnp.bfloat16)


def _pre(nl, nr, el, er, w0, b0, we1, be1c):
    return pl.pallas_call(
        _pre_body,
        out_shape=(
            jax.ShapeDtypeStruct((N_ALL, D2), jnp.float32),
            jax.ShapeDtypeStruct((DEH, E_ALL), jnp.bfloat16),
        ),
    )(nl, nr, el, er, w0, b0, we1, be1c)


def _msg_body(zt_ref, g_ref, we2t_ref, be2t_ref, msg_ref):
    ewt = jnp.dot(we2t_ref[...], zt_ref[...],
                  preferred_element_type=jnp.float32)
    gt = jnp.transpose(g_ref[:, :D])
    acc = jnp.dot(be2t_ref[...], gt, preferred_element_type=jnp.float32)
    for i in range(D):
        acc = acc + gt[i:i + 1, :] * ewt[i * D:(i + 1) * D, :]
    out = jnp.transpose(acc)
    msg_ref[...] = jnp.concatenate(
        [out, jnp.zeros((ET, D2 - D), jnp.float32)], axis=1)


def _msg(zt, g, we2t, be2t, off):
    blk_off = off // ET
    ne = g.shape[0]
    return pl.pallas_call(
        _msg_body,
        grid=(ne // ET,),
        in_specs=[
            pl.BlockSpec((DEH, ET), lambda i: (0, i + blk_off)),
            pl.BlockSpec((ET, D2), lambda i: (i, 0)),
            pl.BlockSpec((DD, DEH), lambda i: (0, 0)),
            pl.BlockSpec((D, D), lambda i: (0, 0)),
        ],
        out_specs=pl.BlockSpec((ET, D2), lambda i: (i, 0)),
        out_shape=jax.ShapeDtypeStruct((ne, D2), jnp.float32),
    )(zt, g, we2t, be2t)


def _upd_body(pa_ref, pb_ref, h_ref, cb_ref, wm_ref, bm_ref, out_ref):
    agg = (pa_ref[0][:, :D] + pa_ref[1][:, :D]
           + pb_ref[0][:, :D] + pb_ref[1][:, :D])
    h = h_ref[:, :D]
    m = jnp.maximum(agg + h + cb_ref[...], 0.0)
    hn = (
        jnp.dot(m, wm_ref[0], preferred_element_type=jnp.float32)
        + jnp.dot(h, wm_ref[1], preferred_element_type=jnp.float32)
        + bm_ref[...]
    )
    out_ref[...] = jnp.concatenate(
        [hn, jnp.zeros((N_ALL, D2 - D), jnp.float32)], axis=1)


def _update(parts_a, parts_b, h, cb, wm2, bm):
    return pl.pallas_call(
        _upd_body,
        out_shape=jax.ShapeDtypeStruct((N_ALL, D2), jnp.float32),
        input_output_aliases={2: 0},
    )(parts_a, parts_b, h, cb, wm2, bm)


def _make_att(q0, q1, k0, k1, flip=False):
    def _att_body(h_ref, nq_ref, nk_ref, mask_ref,
                  lnq_g_ref, lnq_b_ref, lnk_g_ref, lnk_b_ref,
                  wq_ref, wk_ref, wv_ref,
                  w1_ref, b1_ref, g1_ref, bn1_ref,
                  w2_ref, b2_ref, g2_ref, bn2_ref,
                  out_ref):
        hq = _ln(h_ref[q0:q1, :D] + nq_ref[...], lnq_g_ref[...],
                 lnq_b_ref[...])
        hk = _ln(h_ref[k0:k1, :D] + nk_ref[...], lnk_g_ref[...],
                 lnk_b_ref[...])
        q = _lrelu(jnp.dot(hq, wq_ref[...],
                           preferred_element_type=jnp.float32))
        k = _lrelu(jnp.dot(hk, wk_ref[...],
                           preferred_element_type=jnp.float32))
        v = jnp.dot(hk, wv_ref[...], preferred_element_type=jnp.float32)
        mask = mask_ref[...]
        if flip:
            s = lax.dot_general(k, q, (((1,), (1,)), ((), ())),
                                preferred_element_type=jnp.float32)
            a = mask * s - 1000.0 * (1.0 - mask)
            a = a - jnp.max(a, axis=0, keepdims=True)
            e = jnp.exp(a)
            prob = e / jnp.sum(e, axis=0, keepdims=True)
            att = lax.dot_general(prob, v, (((0,), (0,)), ((), ())),
                                  preferred_element_type=jnp.float32)
        else:
            s = lax.dot_general(q, k, (((1,), (1,)), ((), ())),
                                preferred_element_type=jnp.float32)
            a = mask * s - 1000.0 * (1.0 - mask)
            a = a - jnp.max(a, axis=1, keepdims=True)
            e = jnp.exp(a)
            prob = e / jnp.sum(e, axis=1, keepdims=True)
            att = jnp.dot(prob, v, preferred_element_type=jnp.float32)
        x = hq + att
        y = _lrelu(_ln(jnp.dot(x, w1_ref[...],
                               preferred_element_type=jnp.float32)
                       + b1_ref[...], g1_ref[...], bn1_ref[...]))
        out_ref[...] = _ln(jnp.dot(y, w2_ref[...],
                                   preferred_element_type=jnp.float32)
                           + b2_ref[...], g2_ref[...], bn2_ref[...])

    def call(h, nq, nk, mask, lnq_g, lnq_b, lnk_g, lnk_b, wq, wk, wv, mp):
        r1 = lambda a: a.reshape(1, -1)
        return pl.pallas_call(
            _att_body,
            out_shape=jax.ShapeDtypeStruct((q1 - q0, D), jnp.float32),
        )(h, nq, nk, mask,
          r1(lnq_g), r1(lnq_b), r1(lnk_g), r1(lnk_b),
          wq, wk, wv,
          mp['W1'], r1(mp['b1']), r1(mp['g1']), r1(mp['bn1']),
          mp['W2'], r1(mp['b2']), r1(mp['g2']), r1(mp['bn2']))

    return call


_att_lig = _make_att(0, N_LIG, N_LIG, N_REAL)
_att_rec = _make_att(N_LIG, N_REAL, 0, N_LIG)


def _sc_mesh():
    return plsc.VectorSubcoreMesh(core_axis_name="c", subcore_axis_name="s",
                                  num_cores=NC, num_subcores=NS)


def _make_gather(nchunk):
    epw = nchunk * CHUNK

    def body(h_hbm, idx_hbm, out_hbm, idx_v, rows_v, sem):
        wid = lax.axis_index("s") * NC + lax.axis_index("c")
        pltpu.sync_copy(idx_hbm.at[wid], idx_v)
        copies = []
        for ci in range(nchunk):
            copies.append(pltpu.async_copy(
                h_hbm.at[idx_v.at[ci]],
                rows_v.at[pl.ds(ci * CHUNK, CHUNK)], sem))
        for cp in copies:
            cp.wait()
        pltpu.sync_copy(rows_v, out_hbm.at[pl.ds(wid * epw, epw)])

    def call(h, idx3):
        return pl.kernel(
            body,
            out_type=jax.ShapeDtypeStruct((NW * epw, D2), jnp.float32),
            mesh=_sc_mesh(),
            scratch_types=[
                pltpu.VMEM((nchunk, CHUNK), jnp.int32),
                pltpu.VMEM((epw, D2), jnp.float32),
                pltpu.SemaphoreType.DMA,
            ],
        )(h, idx3)

    return call


def _make_scatter(nchunk):
    epw = nchunk * CHUNK
    zrows = min(epw, RPS)

    def body(msg_hbm, dst_hbm, out_hbm, idx_v, rows_v, accum, sem):
        cid = lax.axis_index("c")
        sid = lax.axis_index("s")
        wid = sid * NC + cid
        pltpu.sync_copy(dst_hbm.at[wid], idx_v)
        zeros16 = jnp.zeros((16,), jnp.float32)

        @pl.loop(0, zrows)
        def _(i):
            for j in range(D2 // 16):
                rows_v[i, pl.ds(j * 16, 16)] = zeros16

        for t in range(RPS // zrows):
            pltpu.sync_copy(rows_v.at[pl.ds(0, zrows)],
                            accum.at[pl.ds(sid * RPS + t * zrows, zrows)])
        pltpu.sync_copy(msg_hbm.at[pl.ds(wid * epw, epw)], rows_v)
        plsc.subcore_barrier()
        for ci in range(nchunk):
            pltpu.sync_copy(rows_v.at[pl.ds(ci * CHUNK, CHUNK)],
                            accum.at[idx_v.at[ci]], add=True)
        plsc.subcore_barrier()
        pltpu.sync_copy(accum.at[pl.ds(sid * RPS, RPS)],
                        out_hbm.at[cid].at[pl.ds(sid * RPS, RPS)])

    def call(msg, dst3):
        return pl.kernel(
            body,
            out_type=jax.ShapeDtypeStruct((NC, N_ALL, D2), jnp.float32),
            mesh=_sc_mesh(),
            scratch_types=[
                pltpu.VMEM((nchunk, CHUNK), jnp.int32),
                pltpu.VMEM((epw, D2), jnp.float32),
                pltpu.VMEM_SHARED((N_ALL, D2), jnp.float32),
                pltpu.SemaphoreType.DMA,
            ],
        )(msg, dst3)

    return call


_GATHERS = {NCHA: _make_gather(NCHA), NCHB: _make_gather(NCHB)}
_SCATTERS = {NCHA: _make_scatter(NCHA), NCHB: _make_scatter(NCHB)}


def _sc_gather(h, idx3):
    return _GATHERS[idx3.shape[1]](h, idx3)


def _sc_scatter(msg, dst3):
    return _SCATTERS[dst3.shape[1]](msg, dst3)


def kernel(lig_n_feat, lig_e_feat, rec_n_feat, rec_e_feat, mask, params,
           lig_edge_index, rec_edge_index):
    p = params

    pad_idx = jnp.full((E_ALL - E_REAL,), PAD_ROW, jnp.int32)
    src = jnp.concatenate([lig_edge_index[0], rec_edge_index[0] + N_LIG,
                           pad_idx])
    dst = jnp.concatenate([lig_edge_index[1], rec_edge_index[1] + N_LIG,
                           pad_idx])
    src3a = src[:EA].reshape(NW, NCHA, CHUNK)
    src3b = src[EA:].reshape(NW, NCHB, CHUNK)
    dst3a = dst[:EA].reshape(NW, NCHA, CHUNK)
    dst3b = dst[EA:].reshape(NW, NCHB, CHUNK)

    r1 = lambda a: a.reshape(1, -1)
    wm2 = p['Wm'].reshape(2, D, D)
    we2t = p['We2'].T.astype(jnp.bfloat16)
    be2t = p['be2'].reshape(D, D).T

    h, zt = _pre(lig_n_feat, rec_n_feat, lig_e_feat, rec_e_feat,
                 p['W0'], r1(p['b0']), p['We1'], p['be1'].reshape(-1, 1))
    for _ in range(STEPS):
        g_a = _sc_gather(h, src3a)
        g_b = _sc_gather(h, src3b)
        msg_a = _msg(zt, g_a, we2t, be2t, 0)
        msg_b = _msg(zt, g_b, we2t, be2t, EA)
        parts_a = _sc_scatter(msg_a, dst3a)
        parts_b = _sc_scatter(msg_b, dst3b)
        h = _update(parts_a, parts_b, h, r1(p['conv_bias']), wm2, r1(p['bm']))

    out_lig = _att_lig(h, lig_n_feat, rec_n_feat, mask,
                       p['ln_lig_g'], p['ln_lig_b'], p['ln_rec_g'],
                       p['ln_rec_b'], p['Q_lig'], p['K_rec'], p['V_rec'],
                       p['mlp_lig'])
    out_rec = _att_rec(h, rec_n_feat, lig_n_feat, mask.T,
                       p['ln_rec_g'], p['ln_rec_b'], p['ln_lig_g'],
                       p['ln_lig_b'], p['Q_rec'], p['K_lig'], p['V_lig'],
                       p['mlp_rec'])
    return out_lig, out_rec
